# baseline (speedup 1.0000x reference)
import functools

import jax
import jax.numpy as jnp
import numpy as np
from jax import lax
from jax.experimental import pallas as pl
from jax.experimental.pallas import tpu as pltpu
from jax.experimental.pallas import tpu_sc as plsc

E = 8
M = 1024
F = 4096
N = 4096
C = 512
TB = 1024
NB = N // TB
FD = 1024
FB = F // FD

NW = 32
TPW = N // NW
RCH = 32
NCH = TPW // RCH

_TRI = np.tril(np.ones((TB, TB), np.float32), -1)


def _gate_body(tri_ref, x_ref, wg_ref,
               idxd_ref, idxc_ref, scale_ref, cnt_ref, laux_ref,
               cnt_acc, me_acc):
    b = pl.program_id(0)

    @pl.when(b == 0)
    def _init():
        cnt_acc[...] = jnp.zeros_like(cnt_acc)
        me_acc[...] = jnp.zeros_like(me_acc)

    x = x_ref[...]
    logits = lax.dot_general(x.astype(jnp.bfloat16),
                             wg_ref[...].astype(jnp.bfloat16),
                             (((1,), (1,)), ((), ())),
                             preferred_element_type=jnp.float32)
    mx = jnp.max(logits, axis=1, keepdims=True)
    lane = lax.broadcasted_iota(jnp.int32, (TB, E), 1)
    idx = jnp.min(jnp.where(logits == mx, lane, E), axis=1).astype(jnp.int32)
    eg = jnp.exp(logits - mx)
    gates = eg / jnp.sum(eg, axis=1, keepdims=True)
    mask = (lane == idx[:, None]).astype(jnp.float32)
    gate_s = jnp.sum(gates * mask, axis=1)

    carry = cnt_acc[...]
    locs = lax.dot(tri_ref[...], mask.astype(jnp.bfloat16),
                   preferred_element_type=jnp.float32) + carry
    loc = jnp.sum(locs * mask, axis=1).astype(jnp.int32)
    cnt_acc[...] = carry + jnp.sum(mask, axis=0, keepdims=True)
    me_acc[...] = me_acc[...] + jnp.sum(gates, axis=0, keepdims=True)

    valid = loc < C
    flat = idx * C + loc
    idxd_ref[...] = jnp.where(valid, flat, N).reshape(1, 1, TB)
    idxc_ref[...] = jnp.where(valid, flat, 0).reshape(1, 1, TB)
    scale = gate_s * valid.astype(jnp.float32)
    scale_ref[...] = jnp.broadcast_to(scale[:, None], (TB, 128))

    @pl.when(b == NB - 1)
    def _fin():
        cnt = cnt_acc[...]
        lane8 = lax.broadcasted_iota(jnp.int32, (1, E), 1)
        cand = jnp.where(cnt < C, (lane8 * C + (C - 1)).astype(jnp.float32),
                         jnp.float32(2 ** 30))
        empty = jnp.min(cand).reshape(1, 1)
        cnt_ref[...] = jnp.concatenate(
            [cnt, empty, jnp.zeros((1, 15 - E), jnp.float32)], axis=1)
        laux = jnp.sum(me_acc[...] * cnt) * (E / (N * N))
        laux_ref[...] = jnp.broadcast_to(laux, (1, 1))


def _gate(xr, wg):
    tri = jnp.asarray(_TRI, dtype=jnp.bfloat16)
    return pl.pallas_call(
        _gate_body,
        grid=(NB,),
        in_specs=[
            pl.BlockSpec((TB, TB), lambda b: (0, 0)),
            pl.BlockSpec((TB, M), lambda b: (b, 0)),
            pl.BlockSpec((E, M), lambda b: (0, 0)),
        ],
        out_specs=[
            pl.BlockSpec((1, 1, TB), lambda b: (b, 0, 0)),
            pl.BlockSpec((1, 1, TB), lambda b: (b, 0, 0)),
            pl.BlockSpec((TB, 128), lambda b: (b, 0)),
            pl.BlockSpec((1, 16), lambda b: (0, 0)),
            pl.BlockSpec((1, 1), lambda b: (0, 0)),
        ],
        out_shape=[
            jax.ShapeDtypeStruct((NB, 1, TB), jnp.int32),
            jax.ShapeDtypeStruct((NB, 1, TB), jnp.int32),
            jax.ShapeDtypeStruct((N, 128), jnp.float32),
            jax.ShapeDtypeStruct((1, 16), jnp.float32),
            jax.ShapeDtypeStruct((1, 1), jnp.float32),
        ],
        scratch_shapes=[
            pltpu.VMEM((1, E), jnp.float32),
            pltpu.VMEM((1, E), jnp.float32),
        ],
        compiler_params=pltpu.CompilerParams(
            dimension_semantics=("arbitrary",)),
    )(tri, xr, wg)


_SC_MESH = plsc.VectorSubcoreMesh(core_axis_name="c", subcore_axis_name="s")


@functools.partial(
    pl.kernel,
    mesh=_SC_MESH,
    out_type=[
        jax.ShapeDtypeStruct((N + 8, M), jnp.float32),
        jax.ShapeDtypeStruct((N + 8, 128), jnp.float32),
        jax.ShapeDtypeStruct((NW, NCH, RCH), jnp.int32),
    ],
    scratch_types=[
        pltpu.VMEM((NCH, RCH), jnp.int32),
        pltpu.VMEM((NCH, RCH), jnp.int32),
        pltpu.VMEM((TPW, 128), jnp.float32),
        pltpu.VMEM((16,), jnp.float32),
        pltpu.VMEM((RCH, M), jnp.float32),
        pltpu.VMEM((RCH, M), jnp.float32),
        pltpu.SemaphoreType.DMA,
        pltpu.SemaphoreType.DMA,
        pltpu.SemaphoreType.DMA,
        pltpu.SemaphoreType.DMA,
        pltpu.SemaphoreType.DMA,
    ],
)
def _dispatch(x_hbm, idxd_hbm, idxc_hbm, scl_hbm, cnt_hbm,
              disp_hbm, sslot_hbm, idxf_hbm,
              idx_v, idc_v, scl_v, cnt_v, buf0, buf1,
              l0, l1, s0, s1, ssc):
    bufs = [buf0, buf1]
    lsems = [l0, l1]
    ssems = [s0, s1]
    wid = lax.axis_index("s") * 2 + lax.axis_index("c")
    base = wid * TPW
    pltpu.sync_copy(idxd_hbm.at[wid], idx_v)
    pltpu.sync_copy(idxc_hbm.at[wid], idc_v)
    pltpu.sync_copy(scl_hbm.at[wid], scl_v)
    pltpu.sync_copy(cnt_hbm.at[0], cnt_v)

    empty = cnt_v[...][E].astype(jnp.int32)
    for ch in range(NCH):
        for j in range(RCH // 16):
            sl = pl.ds(j * 16, 16)
            d = idx_v[ch, sl]
            v = idc_v[ch, sl]
            idc_v[ch, sl] = jnp.where(d == N, empty, v)
    pltpu.sync_copy(idc_v, idxf_hbm.at[wid])

    sscats = []
    for ch in range(NCH):
        sscats.append(pltpu.async_copy(
            scl_v.at[pl.ds(ch * RCH, RCH)], sslot_hbm.at[idx_v.at[ch]], ssc))

    loads = {0: pltpu.async_copy(x_hbm.at[pl.ds(base, RCH)], buf0, l0)}
    scats = {}
    for ch in range(NCH):
        b = ch % 2
        nb = (ch + 1) % 2
        if ch + 1 < NCH:
            if ch - 1 >= 0:
                scats[ch - 1].wait()
            loads[ch + 1] = pltpu.async_copy(
                x_hbm.at[pl.ds(base + (ch + 1) * RCH, RCH)], bufs[nb],
                lsems[nb])
        loads[ch].wait()
        scats[ch] = pltpu.async_copy(bufs[b], disp_hbm.at[idx_v.at[ch]],
                                     ssems[b])
    scats[NCH - 2].wait()
    scats[NCH - 1].wait()
    for h in sscats:
        h.wait()


def _ffn_body(cnt_ref, x_ref, w1_ref, w2_ref, ss_ref, out_ref, acc):
    e = pl.program_id(0)
    f = pl.program_id(1)
    cnt = cnt_ref[0, e].astype(jnp.int32)
    row = lax.broadcasted_iota(jnp.int32, (C, 1), 0)
    rmask = (row < cnt).astype(jnp.float32)
    x = x_ref[...] * rmask
    h = jnp.maximum(
        lax.dot(x.astype(jnp.bfloat16), w1_ref[0].astype(jnp.bfloat16),
                preferred_element_type=jnp.float32), 0.0)
    p = lax.dot(h.astype(jnp.bfloat16), w2_ref[0].astype(jnp.bfloat16),
                preferred_element_type=jnp.float32)

    @pl.when(f == 0)
    def _first():
        acc[...] = p

    @pl.when(f != 0)
    def _rest():
        acc[...] = acc[...] + p

    @pl.when(f == FB - 1)
    def _fin():
        s = ss_ref[...][:, :1] * rmask
        out_ref[...] = acc[...] * s


def _ffn(cnt, disp, W1, W2, sslot):
    return pl.pallas_call(
        _ffn_body,
        grid=(E, FB),
        in_specs=[
            pl.BlockSpec(memory_space=pltpu.SMEM),
            pl.BlockSpec((C, M), lambda e, f: (e, 0)),
            pl.BlockSpec((1, M, FD), lambda e, f: (e, 0, f)),
            pl.BlockSpec((1, FD, M), lambda e, f: (e, f, 0)),
            pl.BlockSpec((C, 128), lambda e, f: (e, 0)),
        ],
        out_specs=pl.BlockSpec((C, M), lambda e, f: (e, 0)),
        out_shape=jax.ShapeDtypeStruct((N, M), jnp.float32),
        scratch_shapes=[pltpu.VMEM((C, M), jnp.float32)],
        compiler_params=pltpu.CompilerParams(
            dimension_semantics=("arbitrary", "arbitrary")),
    )(cnt, disp, W1, W2, sslot)


@functools.partial(
    pl.kernel,
    mesh=_SC_MESH,
    out_type=jax.ShapeDtypeStruct((N, M), jnp.float32),
    scratch_types=[
        pltpu.VMEM((NCH, RCH), jnp.int32),
        pltpu.VMEM((RCH, M), jnp.float32),
        pltpu.VMEM((RCH, M), jnp.float32),
        pltpu.SemaphoreType.DMA,
        pltpu.SemaphoreType.DMA,
        pltpu.SemaphoreType.DMA,
        pltpu.SemaphoreType.DMA,
    ],
)
def _combine(eo_hbm, idx_hbm, out_hbm, idx_v, buf0, buf1, g0, g1, s0, s1):
    bufs = [buf0, buf1]
    gsems = [g0, g1]
    ssems = [s0, s1]
    wid = lax.axis_index("s") * 2 + lax.axis_index("c")
    base = wid * TPW
    pltpu.sync_copy(idx_hbm.at[wid], idx_v)
    gathers = {0: pltpu.async_copy(eo_hbm.at[idx_v.at[0]], buf0, g0)}
    stores = {}
    for ch in range(NCH):
        b = ch % 2
        nb = (ch + 1) % 2
        if ch + 1 < NCH:
            if ch - 1 >= 0:
                stores[ch - 1].wait()
            gathers[ch + 1] = pltpu.async_copy(
                eo_hbm.at[idx_v.at[ch + 1]], bufs[nb], gsems[nb])
        gathers[ch].wait()
        stores[ch] = pltpu.async_copy(
            bufs[b], out_hbm.at[pl.ds(base + ch * RCH, RCH)], ssems[b])
    stores[NCH - 2].wait()
    stores[NCH - 1].wait()


def kernel(x, wg, W1, W2):
    S0, T0, _ = x.shape
    xr = x.reshape(N, M)
    idxd3, idxc3, scale_b, cnt, laux = _gate(xr, wg)
    idxd = idxd3.reshape(NW, NCH, RCH)
    idxc = idxc3.reshape(NW, NCH, RCH)
    disp, sslot, idxf = _dispatch(xr, idxd, idxc,
                                  scale_b.reshape(NW, TPW, 128), cnt)
    eo = _ffn(cnt, disp, W1, W2, sslot)
    combined = _combine(eo, idxf).reshape(S0, T0, M)
    return combined, laux.reshape(())


# speedup vs baseline: 1.0068x; 1.0068x over previous
import functools

import jax
import jax.numpy as jnp
import numpy as np
from jax import lax
from jax.experimental import pallas as pl
from jax.experimental.pallas import tpu as pltpu
from jax.experimental.pallas import tpu_sc as plsc

E = 8
M = 1024
F = 4096
N = 4096
C = 512
TB = 1024
NB = N // TB
FD = 1024
FB = F // FD

NW = 32
TPW = N // NW
RCH = 32
NCH = TPW // RCH

_TRI = np.tril(np.ones((TB, TB), np.float32), -1)


def _gate_body(tri_ref, x_ref, wg_ref,
               idxd_ref, idxc_ref, scale_ref, cnt_ref, laux_ref,
               cnt_acc, me_acc):
    b = pl.program_id(0)

    @pl.when(b == 0)
    def _init():
        cnt_acc[...] = jnp.zeros_like(cnt_acc)
        me_acc[...] = jnp.zeros_like(me_acc)

    x = x_ref[...]
    logits = lax.dot_general(x.astype(jnp.bfloat16),
                             wg_ref[...].astype(jnp.bfloat16),
                             (((1,), (1,)), ((), ())),
                             preferred_element_type=jnp.float32)
    mx = jnp.max(logits, axis=1, keepdims=True)
    lane = lax.broadcasted_iota(jnp.int32, (TB, E), 1)
    idx = jnp.min(jnp.where(logits == mx, lane, E), axis=1).astype(jnp.int32)
    eg = jnp.exp(logits - mx)
    gates = eg / jnp.sum(eg, axis=1, keepdims=True)
    mask = (lane == idx[:, None]).astype(jnp.float32)
    gate_s = jnp.sum(gates * mask, axis=1)

    carry = cnt_acc[...]
    locs = lax.dot(tri_ref[...], mask.astype(jnp.bfloat16),
                   preferred_element_type=jnp.float32) + carry
    loc = jnp.sum(locs * mask, axis=1).astype(jnp.int32)
    cnt_acc[...] = carry + jnp.sum(mask, axis=0, keepdims=True)
    me_acc[...] = me_acc[...] + jnp.sum(gates, axis=0, keepdims=True)

    valid = loc < C
    flat = idx * C + loc
    idxd_ref[...] = jnp.where(valid, flat, N).reshape(1, 1, TB)
    idxc_ref[...] = jnp.where(valid, flat, 0).reshape(1, 1, TB)
    scale = gate_s * valid.astype(jnp.float32)
    scale_ref[...] = jnp.broadcast_to(scale[:, None], (TB, 128))

    @pl.when(b == NB - 1)
    def _fin():
        cnt = cnt_acc[...]
        lane8 = lax.broadcasted_iota(jnp.int32, (1, E), 1)
        cand = jnp.where(cnt < C, (lane8 * C + (C - 1)).astype(jnp.float32),
                         jnp.float32(2 ** 30))
        empty = jnp.min(cand).reshape(1, 1)
        cnt_ref[...] = jnp.concatenate(
            [cnt, empty, jnp.zeros((1, 15 - E), jnp.float32)], axis=1)
        laux = jnp.sum(me_acc[...] * cnt) * (E / (N * N))
        laux_ref[...] = jnp.broadcast_to(laux, (1, 1))


def _gate(xr, wg):
    tri = jnp.asarray(_TRI, dtype=jnp.bfloat16)
    return pl.pallas_call(
        _gate_body,
        grid=(NB,),
        in_specs=[
            pl.BlockSpec((TB, TB), lambda b: (0, 0)),
            pl.BlockSpec((TB, M), lambda b: (b, 0)),
            pl.BlockSpec((E, M), lambda b: (0, 0)),
        ],
        out_specs=[
            pl.BlockSpec((1, 1, TB), lambda b: (b, 0, 0)),
            pl.BlockSpec((1, 1, TB), lambda b: (b, 0, 0)),
            pl.BlockSpec((TB, 128), lambda b: (b, 0)),
            pl.BlockSpec((1, 16), lambda b: (0, 0)),
            pl.BlockSpec((1, 1), lambda b: (0, 0)),
        ],
        out_shape=[
            jax.ShapeDtypeStruct((NB, 1, TB), jnp.int32),
            jax.ShapeDtypeStruct((NB, 1, TB), jnp.int32),
            jax.ShapeDtypeStruct((N, 128), jnp.float32),
            jax.ShapeDtypeStruct((1, 16), jnp.float32),
            jax.ShapeDtypeStruct((1, 1), jnp.float32),
        ],
        scratch_shapes=[
            pltpu.VMEM((1, E), jnp.float32),
            pltpu.VMEM((1, E), jnp.float32),
        ],
        compiler_params=pltpu.CompilerParams(
            dimension_semantics=("arbitrary",)),
    )(tri, xr, wg)


_SC_MESH = plsc.VectorSubcoreMesh(core_axis_name="c", subcore_axis_name="s")


@functools.partial(
    pl.kernel,
    mesh=_SC_MESH,
    out_type=[
        jax.ShapeDtypeStruct((N + 8, M), jnp.float32),
        jax.ShapeDtypeStruct((N + 8, 128), jnp.float32),
        jax.ShapeDtypeStruct((NW, NCH, RCH), jnp.int32),
    ],
    scratch_types=[
        pltpu.VMEM((NCH, RCH), jnp.int32),
        pltpu.VMEM((TPW,), jnp.int32),
        pltpu.VMEM((NCH, RCH), jnp.int32),
        pltpu.VMEM((TPW, 128), jnp.float32),
        pltpu.VMEM((16,), jnp.float32),
        pltpu.VMEM((RCH, M), jnp.float32),
        pltpu.VMEM((RCH, M), jnp.float32),
        pltpu.SemaphoreType.DMA,
        pltpu.SemaphoreType.DMA,
        pltpu.SemaphoreType.DMA,
        pltpu.SemaphoreType.DMA,
        pltpu.SemaphoreType.DMA,
        pltpu.SemaphoreType.DMA,
    ],
)
def _dispatch(x_hbm, idxd_hbm, idxdf_hbm, idxc_hbm, scl_hbm, cnt_hbm,
              disp_hbm, sslot_hbm, idxf_hbm,
              idx_v, idxf_v, idc_v, scl_v, cnt_v, buf0, buf1,
              l0, l1, s0, s1, ssc, pre):
    bufs = [buf0, buf1]
    lsems = [l0, l1]
    ssems = [s0, s1]
    wid = lax.axis_index("s") * 2 + lax.axis_index("c")
    base = wid * TPW
    p0 = pltpu.async_copy(idxd_hbm.at[wid], idx_v, pre)
    p1 = pltpu.async_copy(idxdf_hbm.at[wid], idxf_v, pre)
    p2 = pltpu.async_copy(idxc_hbm.at[wid], idc_v, pre)
    p3 = pltpu.async_copy(scl_hbm.at[wid], scl_v, pre)
    p4 = pltpu.async_copy(cnt_hbm.at[0], cnt_v, pre)
    for p in (p0, p1, p2, p3, p4):
        p.wait()

    sscat = pltpu.async_copy(scl_v, sslot_hbm.at[idxf_v], ssc)

    empty = cnt_v[...][E].astype(jnp.int32)
    for ch in range(NCH):
        for j in range(RCH // 16):
            sl = pl.ds(j * 16, 16)
            d = idx_v[ch, sl]
            v = idc_v[ch, sl]
            idc_v[ch, sl] = jnp.where(d == N, empty, v)
    pltpu.sync_copy(idc_v, idxf_hbm.at[wid])

    loads = {0: pltpu.async_copy(x_hbm.at[pl.ds(base, RCH)], buf0, l0)}
    scats = {}
    for ch in range(NCH):
        b = ch % 2
        nb = (ch + 1) % 2
        if ch + 1 < NCH:
            if ch - 1 >= 0:
                scats[ch - 1].wait()
            loads[ch + 1] = pltpu.async_copy(
                x_hbm.at[pl.ds(base + (ch + 1) * RCH, RCH)], bufs[nb],
                lsems[nb])
        loads[ch].wait()
        scats[ch] = pltpu.async_copy(bufs[b], disp_hbm.at[idx_v.at[ch]],
                                     ssems[b])
    scats[NCH - 2].wait()
    scats[NCH - 1].wait()
    sscat.wait()


def _ffn_body(cnt_ref, x_ref, w1_ref, w2_ref, ss_ref, out_ref, acc):
    e = pl.program_id(0)
    f = pl.program_id(1)
    cnt = cnt_ref[0, e].astype(jnp.int32)
    row = lax.broadcasted_iota(jnp.int32, (C, 1), 0)
    rmask = (row < cnt).astype(jnp.float32)
    x = x_ref[...] * rmask
    h = jnp.maximum(
        lax.dot(x.astype(jnp.bfloat16), w1_ref[0].astype(jnp.bfloat16),
                preferred_element_type=jnp.float32), 0.0)
    p = lax.dot(h.astype(jnp.bfloat16), w2_ref[0].astype(jnp.bfloat16),
                preferred_element_type=jnp.float32)

    @pl.when(f == 0)
    def _first():
        acc[...] = p

    @pl.when(f != 0)
    def _rest():
        acc[...] = acc[...] + p

    @pl.when(f == FB - 1)
    def _fin():
        s = ss_ref[...][:, :1] * rmask
        out_ref[...] = acc[...] * s


def _ffn(cnt, disp, W1, W2, sslot):
    return pl.pallas_call(
        _ffn_body,
        grid=(E, FB),
        in_specs=[
            pl.BlockSpec(memory_space=pltpu.SMEM),
            pl.BlockSpec((C, M), lambda e, f: (e, 0)),
            pl.BlockSpec((1, M, FD), lambda e, f: (e, 0, f)),
            pl.BlockSpec((1, FD, M), lambda e, f: (e, f, 0)),
            pl.BlockSpec((C, 128), lambda e, f: (e, 0)),
        ],
        out_specs=pl.BlockSpec((C, M), lambda e, f: (e, 0)),
        out_shape=jax.ShapeDtypeStruct((N, M), jnp.float32),
        scratch_shapes=[pltpu.VMEM((C, M), jnp.float32)],
        compiler_params=pltpu.CompilerParams(
            dimension_semantics=("arbitrary", "arbitrary")),
    )(cnt, disp, W1, W2, sslot)


@functools.partial(
    pl.kernel,
    mesh=_SC_MESH,
    out_type=jax.ShapeDtypeStruct((N, M), jnp.float32),
    scratch_types=[
        pltpu.VMEM((NCH, RCH), jnp.int32),
        pltpu.VMEM((RCH, M), jnp.float32),
        pltpu.VMEM((RCH, M), jnp.float32),
        pltpu.SemaphoreType.DMA,
        pltpu.SemaphoreType.DMA,
        pltpu.SemaphoreType.DMA,
        pltpu.SemaphoreType.DMA,
    ],
)
def _combine(eo_hbm, idx_hbm, out_hbm, idx_v, buf0, buf1, g0, g1, s0, s1):
    bufs = [buf0, buf1]
    gsems = [g0, g1]
    ssems = [s0, s1]
    wid = lax.axis_index("s") * 2 + lax.axis_index("c")
    base = wid * TPW
    pltpu.sync_copy(idx_hbm.at[wid], idx_v)
    gathers = {0: pltpu.async_copy(eo_hbm.at[idx_v.at[0]], buf0, g0)}
    stores = {}
    for ch in range(NCH):
        b = ch % 2
        nb = (ch + 1) % 2
        if ch + 1 < NCH:
            if ch - 1 >= 0:
                stores[ch - 1].wait()
            gathers[ch + 1] = pltpu.async_copy(
                eo_hbm.at[idx_v.at[ch + 1]], bufs[nb], gsems[nb])
        gathers[ch].wait()
        stores[ch] = pltpu.async_copy(
            bufs[b], out_hbm.at[pl.ds(base + ch * RCH, RCH)], ssems[b])
    stores[NCH - 2].wait()
    stores[NCH - 1].wait()


def kernel(x, wg, W1, W2):
    S0, T0, _ = x.shape
    xr = x.reshape(N, M)
    idxd3, idxc3, scale_b, cnt, laux = _gate(xr, wg)
    idxd = idxd3.reshape(NW, NCH, RCH)
    idxc = idxc3.reshape(NW, NCH, RCH)
    disp, sslot, idxf = _dispatch(xr, idxd, idxd3.reshape(NW, TPW), idxc,
                                  scale_b.reshape(NW, TPW, 128), cnt)
    eo = _ffn(cnt, disp, W1, W2, sslot)
    combined = _combine(eo, idxf).reshape(S0, T0, M)
    return combined, laux.reshape(())


# speedup vs baseline: 1.0676x; 1.0604x over previous
import functools

import jax
import jax.numpy as jnp
import numpy as np
from jax import lax
from jax.experimental import pallas as pl
from jax.experimental.pallas import tpu as pltpu
from jax.experimental.pallas import tpu_sc as plsc

E = 8
M = 1024
F = 4096
N = 4096
C = 512
TB = 512
NB = N // TB
FD = 2048
FB = F // FD

NW = 32
TPW = N // NW
RCH = 32
NCH = TPW // RCH

_TRI = np.tril(np.ones((TB, TB), np.float32), -1)


def _gate_body(tri_ref, x_ref, wg_ref,
               idxd_ref, idxc_ref, scale_ref, cnt_ref, laux_ref,
               cnt_acc, me_acc):
    b = pl.program_id(0)

    @pl.when(b == 0)
    def _init():
        cnt_acc[...] = jnp.zeros_like(cnt_acc)
        me_acc[...] = jnp.zeros_like(me_acc)

    x = x_ref[...]
    logits = lax.dot_general(x.astype(jnp.bfloat16),
                             wg_ref[...].astype(jnp.bfloat16),
                             (((1,), (1,)), ((), ())),
                             preferred_element_type=jnp.float32)
    mx = jnp.max(logits, axis=1, keepdims=True)
    lane = lax.broadcasted_iota(jnp.int32, (TB, E), 1)
    idx = jnp.min(jnp.where(logits == mx, lane, E), axis=1).astype(jnp.int32)
    eg = jnp.exp(logits - mx)
    gates = eg / jnp.sum(eg, axis=1, keepdims=True)
    mask = (lane == idx[:, None]).astype(jnp.float32)
    gate_s = jnp.sum(gates * mask, axis=1)

    carry = cnt_acc[...]
    locs = lax.dot(tri_ref[...], mask.astype(jnp.bfloat16),
                   preferred_element_type=jnp.float32) + carry
    loc = jnp.sum(locs * mask, axis=1).astype(jnp.int32)
    cnt_acc[...] = carry + jnp.sum(mask, axis=0, keepdims=True)
    me_acc[...] = me_acc[...] + jnp.sum(gates, axis=0, keepdims=True)

    valid = loc < C
    flat = idx * C + loc
    idxd_ref[...] = jnp.where(valid, flat, N).reshape(1, 1, TB)
    idxc_ref[...] = jnp.where(valid, flat, 0).reshape(1, 1, TB)
    scale = gate_s * valid.astype(jnp.float32)
    scale_ref[...] = jnp.broadcast_to(scale[:, None], (TB, 128))

    @pl.when(b == NB - 1)
    def _fin():
        cnt = cnt_acc[...]
        lane8 = lax.broadcasted_iota(jnp.int32, (1, E), 1)
        cand = jnp.where(cnt < C, (lane8 * C + (C - 1)).astype(jnp.float32),
                         jnp.float32(2 ** 30))
        empty = jnp.min(cand).reshape(1, 1)
        cnt_ref[...] = jnp.concatenate(
            [cnt, empty, jnp.zeros((1, 15 - E), jnp.float32)], axis=1)
        laux = jnp.sum(me_acc[...] * cnt) * (E / (N * N))
        laux_ref[...] = jnp.broadcast_to(laux, (1, 1))


def _gate(xr, wg):
    tri = jnp.asarray(_TRI, dtype=jnp.bfloat16)
    return pl.pallas_call(
        _gate_body,
        grid=(NB,),
        in_specs=[
            pl.BlockSpec((TB, TB), lambda b: (0, 0)),
            pl.BlockSpec((TB, M), lambda b: (b, 0)),
            pl.BlockSpec((E, M), lambda b: (0, 0)),
        ],
        out_specs=[
            pl.BlockSpec((1, 1, TB), lambda b: (b, 0, 0)),
            pl.BlockSpec((1, 1, TB), lambda b: (b, 0, 0)),
            pl.BlockSpec((TB, 128), lambda b: (b, 0)),
            pl.BlockSpec((1, 16), lambda b: (0, 0)),
            pl.BlockSpec((1, 1), lambda b: (0, 0)),
        ],
        out_shape=[
            jax.ShapeDtypeStruct((NB, 1, TB), jnp.int32),
            jax.ShapeDtypeStruct((NB, 1, TB), jnp.int32),
            jax.ShapeDtypeStruct((N, 128), jnp.float32),
            jax.ShapeDtypeStruct((1, 16), jnp.float32),
            jax.ShapeDtypeStruct((1, 1), jnp.float32),
        ],
        scratch_shapes=[
            pltpu.VMEM((1, E), jnp.float32),
            pltpu.VMEM((1, E), jnp.float32),
        ],
        compiler_params=pltpu.CompilerParams(
            dimension_semantics=("arbitrary",)),
    )(tri, xr, wg)


_SC_MESH = plsc.VectorSubcoreMesh(core_axis_name="c", subcore_axis_name="s")


@functools.partial(
    pl.kernel,
    mesh=_SC_MESH,
    out_type=[
        jax.ShapeDtypeStruct((N + 8, M), jnp.float32),
        jax.ShapeDtypeStruct((N + 8, 128), jnp.float32),
        jax.ShapeDtypeStruct((NW, NCH, RCH), jnp.int32),
    ],
    scratch_types=[
        pltpu.VMEM((NCH, RCH), jnp.int32),
        pltpu.VMEM((TPW,), jnp.int32),
        pltpu.VMEM((NCH, RCH), jnp.int32),
        pltpu.VMEM((TPW, 128), jnp.float32),
        pltpu.VMEM((16,), jnp.float32),
        pltpu.VMEM((RCH, M), jnp.float32),
        pltpu.VMEM((RCH, M), jnp.float32),
        pltpu.SemaphoreType.DMA,
        pltpu.SemaphoreType.DMA,
        pltpu.SemaphoreType.DMA,
        pltpu.SemaphoreType.DMA,
        pltpu.SemaphoreType.DMA,
        pltpu.SemaphoreType.DMA,
    ],
)
def _dispatch(x_hbm, idxd_hbm, idxdf_hbm, idxc_hbm, scl_hbm, cnt_hbm,
              disp_hbm, sslot_hbm, idxf_hbm,
              idx_v, idxf_v, idc_v, scl_v, cnt_v, buf0, buf1,
              l0, l1, s0, s1, ssc, pre):
    bufs = [buf0, buf1]
    lsems = [l0, l1]
    ssems = [s0, s1]
    wid = lax.axis_index("s") * 2 + lax.axis_index("c")
    base = wid * TPW
    p0 = pltpu.async_copy(idxd_hbm.at[wid], idx_v, pre)
    p1 = pltpu.async_copy(idxdf_hbm.at[wid], idxf_v, pre)
    p2 = pltpu.async_copy(idxc_hbm.at[wid], idc_v, pre)
    p3 = pltpu.async_copy(scl_hbm.at[wid], scl_v, pre)
    p4 = pltpu.async_copy(cnt_hbm.at[0], cnt_v, pre)
    for p in (p0, p1, p2, p3, p4):
        p.wait()

    sscat = pltpu.async_copy(scl_v, sslot_hbm.at[idxf_v], ssc)

    empty = cnt_v[...][E].astype(jnp.int32)
    for ch in range(NCH):
        for j in range(RCH // 16):
            sl = pl.ds(j * 16, 16)
            d = idx_v[ch, sl]
            v = idc_v[ch, sl]
            idc_v[ch, sl] = jnp.where(d == N, empty, v)
    pltpu.sync_copy(idc_v, idxf_hbm.at[wid])

    loads = {0: pltpu.async_copy(x_hbm.at[pl.ds(base, RCH)], buf0, l0)}
    scats = {}
    for ch in range(NCH):
        b = ch % 2
        nb = (ch + 1) % 2
        if ch + 1 < NCH:
            if ch - 1 >= 0:
                scats[ch - 1].wait()
            loads[ch + 1] = pltpu.async_copy(
                x_hbm.at[pl.ds(base + (ch + 1) * RCH, RCH)], bufs[nb],
                lsems[nb])
        loads[ch].wait()
        scats[ch] = pltpu.async_copy(bufs[b], disp_hbm.at[idx_v.at[ch]],
                                     ssems[b])
    scats[NCH - 2].wait()
    scats[NCH - 1].wait()
    sscat.wait()


def _ffn_body(cnt_ref, x_ref, w1_ref, w2_ref, ss_ref, out_ref, acc):
    e = pl.program_id(0)
    f = pl.program_id(1)
    cnt = cnt_ref[0, e].astype(jnp.int32)
    row = lax.broadcasted_iota(jnp.int32, (C, 1), 0)
    rmask = (row < cnt).astype(jnp.float32)
    x = x_ref[...] * rmask
    h = jnp.maximum(
        lax.dot(x.astype(jnp.bfloat16), w1_ref[0].astype(jnp.bfloat16),
                preferred_element_type=jnp.float32), 0.0)
    p = lax.dot(h.astype(jnp.bfloat16), w2_ref[0].astype(jnp.bfloat16),
                preferred_element_type=jnp.float32)

    @pl.when(f == 0)
    def _first():
        acc[...] = p

    @pl.when(f != 0)
    def _rest():
        acc[...] = acc[...] + p

    @pl.when(f == FB - 1)
    def _fin():
        s = ss_ref[...][:, :1] * rmask
        out_ref[...] = acc[...] * s


def _ffn(cnt, disp, W1, W2, sslot):
    return pl.pallas_call(
        _ffn_body,
        grid=(E, FB),
        in_specs=[
            pl.BlockSpec(memory_space=pltpu.SMEM),
            pl.BlockSpec((C, M), lambda e, f: (e, 0)),
            pl.BlockSpec((1, M, FD), lambda e, f: (e, 0, f)),
            pl.BlockSpec((1, FD, M), lambda e, f: (e, f, 0)),
            pl.BlockSpec((C, 128), lambda e, f: (e, 0)),
        ],
        out_specs=pl.BlockSpec((C, M), lambda e, f: (e, 0)),
        out_shape=jax.ShapeDtypeStruct((N, M), jnp.float32),
        scratch_shapes=[pltpu.VMEM((C, M), jnp.float32)],
        compiler_params=pltpu.CompilerParams(
            dimension_semantics=("arbitrary", "arbitrary")),
    )(cnt, disp, W1, W2, sslot)


@functools.partial(
    pl.kernel,
    mesh=_SC_MESH,
    out_type=jax.ShapeDtypeStruct((N, M), jnp.float32),
    scratch_types=[
        pltpu.VMEM((NCH, RCH), jnp.int32),
        pltpu.VMEM((RCH, M), jnp.float32),
        pltpu.VMEM((RCH, M), jnp.float32),
        pltpu.SemaphoreType.DMA,
        pltpu.SemaphoreType.DMA,
        pltpu.SemaphoreType.DMA,
        pltpu.SemaphoreType.DMA,
    ],
)
def _combine(eo_hbm, idx_hbm, out_hbm, idx_v, buf0, buf1, g0, g1, s0, s1):
    bufs = [buf0, buf1]
    gsems = [g0, g1]
    ssems = [s0, s1]
    wid = lax.axis_index("s") * 2 + lax.axis_index("c")
    base = wid * TPW
    pltpu.sync_copy(idx_hbm.at[wid], idx_v)
    gathers = {0: pltpu.async_copy(eo_hbm.at[idx_v.at[0]], buf0, g0)}
    stores = {}
    for ch in range(NCH):
        b = ch % 2
        nb = (ch + 1) % 2
        if ch + 1 < NCH:
            if ch - 1 >= 0:
                stores[ch - 1].wait()
            gathers[ch + 1] = pltpu.async_copy(
                eo_hbm.at[idx_v.at[ch + 1]], bufs[nb], gsems[nb])
        gathers[ch].wait()
        stores[ch] = pltpu.async_copy(
            bufs[b], out_hbm.at[pl.ds(base + ch * RCH, RCH)], ssems[b])
    stores[NCH - 2].wait()
    stores[NCH - 1].wait()


def kernel(x, wg, W1, W2):
    S0, T0, _ = x.shape
    xr = x.reshape(N, M)
    idxd3, idxc3, scale_b, cnt, laux = _gate(xr, wg)
    idxd = idxd3.reshape(NW, NCH, RCH)
    idxc = idxc3.reshape(NW, NCH, RCH)
    disp, sslot, idxf = _dispatch(xr, idxd, idxd3.reshape(NW, TPW), idxc,
                                  scale_b.reshape(NW, TPW, 128), cnt)
    eo = _ffn(cnt, disp, W1, W2, sslot)
    combined = _combine(eo, idxf).reshape(S0, T0, M)
    return combined, laux.reshape(())


# speedup vs baseline: 1.0805x; 1.0120x over previous
import functools

import jax
import jax.numpy as jnp
import numpy as np
from jax import lax
from jax.experimental import pallas as pl
from jax.experimental.pallas import tpu as pltpu
from jax.experimental.pallas import tpu_sc as plsc

E = 8
M = 1024
F = 4096
N = 4096
C = 512
TB = 512
NB = N // TB
FD = 2048
FB = F // FD

NW = 32
TPW = N // NW
RCH = 32
NCH = TPW // RCH

_TRI = np.tril(np.ones((TB, TB), np.float32), -1)


def _gate_body(tri_ref, x_ref, wg_ref,
               idxd_ref, scale_ref, cnt_ref, laux_ref,
               cnt_acc, me_acc):
    b = pl.program_id(0)

    @pl.when(b == 0)
    def _init():
        cnt_acc[...] = jnp.zeros_like(cnt_acc)
        me_acc[...] = jnp.zeros_like(me_acc)

    x = x_ref[...]
    logits = lax.dot_general(x.astype(jnp.bfloat16),
                             wg_ref[...].astype(jnp.bfloat16),
                             (((1,), (1,)), ((), ())),
                             preferred_element_type=jnp.float32)
    mx = jnp.max(logits, axis=1, keepdims=True)
    lane = lax.broadcasted_iota(jnp.int32, (TB, E), 1)
    idx = jnp.min(jnp.where(logits == mx, lane, E), axis=1).astype(jnp.int32)
    eg = jnp.exp(logits - mx)
    gates = eg / jnp.sum(eg, axis=1, keepdims=True)
    mask = (lane == idx[:, None]).astype(jnp.float32)
    gate_s = jnp.sum(gates * mask, axis=1)

    carry = cnt_acc[...]
    locs = lax.dot(tri_ref[...], mask.astype(jnp.bfloat16),
                   preferred_element_type=jnp.float32) + carry
    loc = jnp.sum(locs * mask, axis=1).astype(jnp.int32)
    cnt_acc[...] = carry + jnp.sum(mask, axis=0, keepdims=True)
    me_acc[...] = me_acc[...] + jnp.sum(gates, axis=0, keepdims=True)

    valid = loc < C
    flat = idx * C + loc
    idxd_ref[...] = jnp.where(valid, flat, N).reshape(1, 1, TB)
    scale = gate_s * valid.astype(jnp.float32)
    scale_ref[...] = jnp.broadcast_to(scale[:, None], (TB, 128))

    @pl.when(b == NB - 1)
    def _fin():
        cnt = cnt_acc[...]
        lane8 = lax.broadcasted_iota(jnp.int32, (1, E), 1)
        cand = jnp.where(cnt < C, (lane8 * C + (C - 1)).astype(jnp.float32),
                         jnp.float32(2 ** 30))
        empty = jnp.min(cand).reshape(1, 1)
        cnt_ref[...] = jnp.concatenate(
            [cnt, empty, jnp.zeros((1, 15 - E), jnp.float32)], axis=1)
        laux = jnp.sum(me_acc[...] * cnt) * (E / (N * N))
        laux_ref[...] = jnp.broadcast_to(laux, (1, 1))


def _gate(xr, wg):
    tri = jnp.asarray(_TRI, dtype=jnp.bfloat16)
    return pl.pallas_call(
        _gate_body,
        grid=(NB,),
        in_specs=[
            pl.BlockSpec((TB, TB), lambda b: (0, 0)),
            pl.BlockSpec((TB, M), lambda b: (b, 0)),
            pl.BlockSpec((E, M), lambda b: (0, 0)),
        ],
        out_specs=[
            pl.BlockSpec((1, 1, TB), lambda b: (b, 0, 0)),
            pl.BlockSpec((TB, 128), lambda b: (b, 0)),
            pl.BlockSpec((1, 16), lambda b: (0, 0)),
            pl.BlockSpec((1, 1), lambda b: (0, 0)),
        ],
        out_shape=[
            jax.ShapeDtypeStruct((NB, 1, TB), jnp.int32),
            jax.ShapeDtypeStruct((N, 128), jnp.float32),
            jax.ShapeDtypeStruct((1, 16), jnp.float32),
            jax.ShapeDtypeStruct((1, 1), jnp.float32),
        ],
        scratch_shapes=[
            pltpu.VMEM((1, E), jnp.float32),
            pltpu.VMEM((1, E), jnp.float32),
        ],
        compiler_params=pltpu.CompilerParams(
            dimension_semantics=("arbitrary",)),
    )(tri, xr, wg)


_SC_MESH = plsc.VectorSubcoreMesh(core_axis_name="c", subcore_axis_name="s")


@functools.partial(
    pl.kernel,
    mesh=_SC_MESH,
    out_type=[
        jax.ShapeDtypeStruct((N + 8, M), jnp.float32),
        jax.ShapeDtypeStruct((N + 8, 128), jnp.float32),
        jax.ShapeDtypeStruct((NW, NCH, RCH), jnp.int32),
    ],
    scratch_types=[
        pltpu.VMEM((NCH, RCH), jnp.int32),
        pltpu.VMEM((TPW,), jnp.int32),
        pltpu.VMEM((NCH, RCH), jnp.int32),
        pltpu.VMEM((TPW, 128), jnp.float32),
        pltpu.VMEM((16,), jnp.float32),
        pltpu.VMEM((RCH, M), jnp.float32),
        pltpu.VMEM((RCH, M), jnp.float32),
        pltpu.SemaphoreType.DMA,
        pltpu.SemaphoreType.DMA,
        pltpu.SemaphoreType.DMA,
        pltpu.SemaphoreType.DMA,
        pltpu.SemaphoreType.DMA,
        pltpu.SemaphoreType.DMA,
    ],
)
def _dispatch(x_hbm, idxd_hbm, idxdf_hbm, scl_hbm, cnt_hbm,
              disp_hbm, sslot_hbm, idxf_hbm,
              idx_v, idxf_v, idc_v, scl_v, cnt_v, buf0, buf1,
              l0, l1, s0, s1, ssc, pre):
    bufs = [buf0, buf1]
    lsems = [l0, l1]
    ssems = [s0, s1]
    wid = lax.axis_index("s") * 2 + lax.axis_index("c")
    base = wid * TPW
    p0 = pltpu.async_copy(idxd_hbm.at[wid], idx_v, pre)
    p1 = pltpu.async_copy(idxdf_hbm.at[wid], idxf_v, pre)
    p3 = pltpu.async_copy(scl_hbm.at[wid], scl_v, pre)
    p4 = pltpu.async_copy(cnt_hbm.at[0], cnt_v, pre)
    for p in (p0, p1, p3, p4):
        p.wait()

    sscat = pltpu.async_copy(scl_v, sslot_hbm.at[idxf_v], ssc)

    empty = cnt_v[...][E].astype(jnp.int32)
    for ch in range(NCH):
        for j in range(RCH // 16):
            sl = pl.ds(j * 16, 16)
            d = idx_v[ch, sl]
            idc_v[ch, sl] = jnp.where(d == N, empty, d)
    pltpu.sync_copy(idc_v, idxf_hbm.at[wid])

    loads = {0: pltpu.async_copy(x_hbm.at[pl.ds(base, RCH)], buf0, l0)}
    scats = {}
    for ch in range(NCH):
        b = ch % 2
        nb = (ch + 1) % 2
        if ch + 1 < NCH:
            if ch - 1 >= 0:
                scats[ch - 1].wait()
            loads[ch + 1] = pltpu.async_copy(
                x_hbm.at[pl.ds(base + (ch + 1) * RCH, RCH)], bufs[nb],
                lsems[nb])
        loads[ch].wait()
        scats[ch] = pltpu.async_copy(bufs[b], disp_hbm.at[idx_v.at[ch]],
                                     ssems[b])
    scats[NCH - 2].wait()
    scats[NCH - 1].wait()
    sscat.wait()


def _ffn_body(cnt_ref, x_ref, w1_ref, w2_ref, ss_ref, out_ref, acc):
    e = pl.program_id(0)
    f = pl.program_id(1)
    cnt = cnt_ref[0, e].astype(jnp.int32)
    row = lax.broadcasted_iota(jnp.int32, (C, 1), 0)
    rmask = (row < cnt).astype(jnp.float32)
    x = x_ref[...] * rmask
    h = jnp.maximum(
        lax.dot(x.astype(jnp.bfloat16), w1_ref[0].astype(jnp.bfloat16),
                preferred_element_type=jnp.float32), 0.0)
    p = lax.dot(h.astype(jnp.bfloat16), w2_ref[0].astype(jnp.bfloat16),
                preferred_element_type=jnp.float32)

    @pl.when(f == 0)
    def _first():
        acc[...] = p

    @pl.when(f != 0)
    def _rest():
        acc[...] = acc[...] + p

    @pl.when(f == FB - 1)
    def _fin():
        s = ss_ref[...][:, :1] * rmask
        out_ref[...] = acc[...] * s


def _ffn(cnt, disp, W1, W2, sslot):
    return pl.pallas_call(
        _ffn_body,
        grid=(E, FB),
        in_specs=[
            pl.BlockSpec(memory_space=pltpu.SMEM),
            pl.BlockSpec((C, M), lambda e, f: (e, 0)),
            pl.BlockSpec((1, M, FD), lambda e, f: (e, 0, f)),
            pl.BlockSpec((1, FD, M), lambda e, f: (e, f, 0)),
            pl.BlockSpec((C, 128), lambda e, f: (e, 0)),
        ],
        out_specs=pl.BlockSpec((C, M), lambda e, f: (e, 0)),
        out_shape=jax.ShapeDtypeStruct((N, M), jnp.float32),
        scratch_shapes=[pltpu.VMEM((C, M), jnp.float32)],
        compiler_params=pltpu.CompilerParams(
            dimension_semantics=("arbitrary", "arbitrary")),
    )(cnt, disp, W1, W2, sslot)


@functools.partial(
    pl.kernel,
    mesh=_SC_MESH,
    out_type=jax.ShapeDtypeStruct((N, M), jnp.float32),
    scratch_types=[
        pltpu.VMEM((NCH, RCH), jnp.int32),
        pltpu.VMEM((RCH, M), jnp.float32),
        pltpu.VMEM((RCH, M), jnp.float32),
        pltpu.SemaphoreType.DMA,
        pltpu.SemaphoreType.DMA,
        pltpu.SemaphoreType.DMA,
        pltpu.SemaphoreType.DMA,
    ],
)
def _combine(eo_hbm, idx_hbm, out_hbm, idx_v, buf0, buf1, g0, g1, s0, s1):
    bufs = [buf0, buf1]
    gsems = [g0, g1]
    ssems = [s0, s1]
    wid = lax.axis_index("s") * 2 + lax.axis_index("c")
    base = wid * TPW
    pltpu.sync_copy(idx_hbm.at[wid], idx_v)
    gathers = {0: pltpu.async_copy(eo_hbm.at[idx_v.at[0]], buf0, g0)}
    stores = {}
    for ch in range(NCH):
        b = ch % 2
        nb = (ch + 1) % 2
        if ch + 1 < NCH:
            if ch - 1 >= 0:
                stores[ch - 1].wait()
            gathers[ch + 1] = pltpu.async_copy(
                eo_hbm.at[idx_v.at[ch + 1]], bufs[nb], gsems[nb])
        gathers[ch].wait()
        stores[ch] = pltpu.async_copy(
            bufs[b], out_hbm.at[pl.ds(base + ch * RCH, RCH)], ssems[b])
    stores[NCH - 2].wait()
    stores[NCH - 1].wait()


def kernel(x, wg, W1, W2):
    S0, T0, _ = x.shape
    xr = x.reshape(N, M)
    idxd3, scale_b, cnt, laux = _gate(xr, wg)
    idxd = idxd3.reshape(NW, NCH, RCH)
    disp, sslot, idxf = _dispatch(xr, idxd, idxd3.reshape(NW, TPW),
                                  scale_b.reshape(NW, TPW, 128), cnt)
    eo = _ffn(cnt, disp, W1, W2, sslot)
    combined = _combine(eo, idxf).reshape(S0, T0, M)
    return combined, laux.reshape(())
